# Initial kernel scaffold; baseline (speedup 1.0000x reference)
import functools

import jax
import jax.numpy as jnp
from jax import lax
from jax.experimental import pallas as pl
from jax.experimental.pallas import tpu as pltpu
from jax.experimental.pallas import tpu_sc as plsc

VOCAB = 1000
EMBED = 64
NUM_CORES = 2
NUM_SUBCORES = 16
NUM_WORKERS = NUM_CORES * NUM_SUBCORES

SUB = 128
SUBS_PER_PHASE = 4
PHASE = SUB * SUBS_PER_PHASE


def _sc_gather(x_grp, table):
    _, n_sub, _ = x_grp.shape
    per_w = n_sub * SUB
    n_phase = per_w // PHASE
    n_pair = n_phase // 2
    total = NUM_WORKERS * per_w

    mesh = plsc.VectorSubcoreMesh(
        core_axis_name="c", subcore_axis_name="s",
        num_cores=NUM_CORES, num_subcores=NUM_SUBCORES)

    @functools.partial(
        pl.kernel,
        mesh=mesh,
        out_type=jax.ShapeDtypeStruct((total, EMBED), jnp.float32),
        scratch_types=[
            pltpu.VMEM((n_sub, SUB), jnp.int32),
            pltpu.VMEM((PHASE, EMBED), jnp.float32),
            pltpu.VMEM((PHASE, EMBED), jnp.float32),
            pltpu.SemaphoreType.DMA,
            pltpu.SemaphoreType.DMA,
        ],
    )
    def k(x_hbm, table_hbm, out_hbm, idx_v, buf_a, buf_b, sem_a, sem_b):
        wid = lax.axis_index("s") * NUM_CORES + lax.axis_index("c")
        base_w = wid * per_w

        pltpu.sync_copy(x_hbm.at[wid], idx_v)

        def fire(phase, buf, sem):
            for q in range(SUBS_PER_PHASE):
                sub = phase * SUBS_PER_PHASE + q
                pltpu.async_copy(
                    table_hbm.at[idx_v.at[sub]],
                    buf.at[pl.ds(q * SUB, SUB)],
                    sem)

        def drain_and_store(phase, buf, sem):
            base = base_w + phase * PHASE
            out_slice = out_hbm.at[pl.ds(base, PHASE)]
            pltpu.make_async_copy(out_slice, buf, sem).wait()
            pltpu.sync_copy(buf, out_slice)

        fire(0, buf_a, sem_a)

        def pair(i, carry):
            pa = 2 * i
            fire(pa + 1, buf_b, sem_b)
            drain_and_store(pa, buf_a, sem_a)

            @pl.when(i < n_pair - 1)
            def _():
                fire(pa + 2, buf_a, sem_a)

            drain_and_store(pa + 1, buf_b, sem_b)
            return carry

        lax.fori_loop(0, n_pair, pair, 0)

    return k(x_grp, table)


def kernel(x, embedding_matrix):
    batch, hist = x.shape
    total = batch * hist
    per_w = total // NUM_WORKERS
    x_grp = x.astype(jnp.int32).reshape(NUM_WORKERS, per_w // SUB, SUB)
    out = _sc_gather(x_grp, embedding_matrix)
    return out.reshape(batch, hist, EMBED)


# speedup vs baseline: 5.2490x; 5.2490x over previous
import functools

import jax
import jax.numpy as jnp
from jax import lax
from jax.experimental import pallas as pl
from jax.experimental.pallas import tpu as pltpu
from jax.experimental.pallas import tpu_sc as plsc

VOCAB = 1000
EMBED = 64
NUM_CORES = 2
NUM_SUBCORES = 16
NUM_WORKERS = NUM_CORES * NUM_SUBCORES

SUB = 128
SUBS_PER_PHASE = 4
PHASE = SUB * SUBS_PER_PHASE


def _sc_gather(x_grp, table):
    _, n_sub, _ = x_grp.shape
    per_w = n_sub * SUB
    n_phase = per_w // PHASE
    n_pair = n_phase // 2
    total = NUM_WORKERS * per_w

    mesh = plsc.VectorSubcoreMesh(
        core_axis_name="c", subcore_axis_name="s",
        num_cores=NUM_CORES, num_subcores=NUM_SUBCORES)

    @functools.partial(
        pl.kernel,
        mesh=mesh,
        out_type=jax.ShapeDtypeStruct((total, EMBED), jnp.float32),
        scratch_types=[
            pltpu.VMEM((n_sub, SUB), jnp.int32),
            pltpu.VMEM((PHASE, EMBED), jnp.float32),
            pltpu.VMEM((PHASE, EMBED), jnp.float32),
            pltpu.SemaphoreType.DMA,
            pltpu.SemaphoreType.DMA,
        ],
        compiler_params=pltpu.CompilerParams(use_tc_tiling_on_sc=False),
    )
    def k(x_hbm, table_hbm, out_hbm, idx_v, buf_a, buf_b, sem_a, sem_b):
        wid = lax.axis_index("s") * NUM_CORES + lax.axis_index("c")
        base_w = wid * per_w

        pltpu.sync_copy(x_hbm.at[wid], idx_v)

        def fire(phase, buf, sem):
            for q in range(SUBS_PER_PHASE):
                sub = phase * SUBS_PER_PHASE + q
                pltpu.async_copy(
                    table_hbm.at[idx_v.at[sub]],
                    buf.at[pl.ds(q * SUB, SUB)],
                    sem)

        def drain_and_store(phase, buf, sem):
            base = base_w + phase * PHASE
            out_slice = out_hbm.at[pl.ds(base, PHASE)]
            pltpu.make_async_copy(out_slice, buf, sem).wait()
            pltpu.sync_copy(buf, out_slice)

        fire(0, buf_a, sem_a)

        def pair(i, carry):
            pa = 2 * i
            fire(pa + 1, buf_b, sem_b)
            drain_and_store(pa, buf_a, sem_a)

            @pl.when(i < n_pair - 1)
            def _():
                fire(pa + 2, buf_a, sem_a)

            drain_and_store(pa + 1, buf_b, sem_b)
            return carry

        lax.fori_loop(0, n_pair, pair, 0)

    return k(x_grp, table)


def kernel(x, embedding_matrix):
    batch, hist = x.shape
    total = batch * hist
    per_w = total // NUM_WORKERS
    x_grp = x.astype(jnp.int32).reshape(NUM_WORKERS, per_w // SUB, SUB)
    out = _sc_gather(x_grp, embedding_matrix)
    return out.reshape(batch, hist, EMBED)
